# baseline (speedup 1.0000x reference)
import functools

import jax
import jax.numpy as jnp
from jax import lax
from jax.experimental import pallas as pl
from jax.experimental.pallas import tpu as pltpu
from jax.experimental.pallas import tpu_sc as plsc

B, N, K, D = 4, 4096, 16, 128
BN = B * N
BNK = B * N * K
NW = 32
ROWS_PW = BNK // NW
CH = 512
PB = 128


def _sc_gather_body(tx_hbm, ty_hbm, tz_hbm, feat_hbm, gidx_hbm,
                    px_out, py_out, pz_out, buf_out,
                    txv, tyv, tzv, idxv, sidxv, pxb, pyb, pzb, fbuf,
                    sem_i, sem_f, sem_s):
    wid = lax.axis_index("s") * 2 + lax.axis_index("c")
    base0 = wid * ROWS_PW
    pltpu.sync_copy(tx_hbm, txv)
    pltpu.sync_copy(ty_hbm, tyv)
    pltpu.sync_copy(tz_hbm, tzv)

    def body(j, carry):
        base = base0 + j * CH
        pltpu.async_copy(gidx_hbm.at[pl.ds(base, CH)], idxv, sem_i).wait()
        cp_f = pltpu.async_copy(feat_hbm.at[idxv], fbuf, sem_f)

        def inner(i, c):
            s = pl.ds(i * 16, 16)
            v = idxv[s]
            pxb[s] = plsc.load_gather(txv, [v])
            pyb[s] = plsc.load_gather(tyv, [v])
            pzb[s] = plsc.load_gather(tzv, [v])
            lane = lax.iota(jnp.int32, 16)
            sidxv[s] = (base + i * 16 + lane) * 2 + 1
            return c

        lax.fori_loop(0, CH // 16, inner, 0)
        pltpu.sync_copy(pxb, px_out.at[pl.ds(base, CH)])
        pltpu.sync_copy(pyb, py_out.at[pl.ds(base, CH)])
        pltpu.sync_copy(pzb, pz_out.at[pl.ds(base, CH)])
        cp_f.wait()
        pltpu.async_copy(fbuf, buf_out.at[sidxv], sem_s).wait()
        return carry

    lax.fori_loop(0, ROWS_PW // CH, body, 0)


def _sc_gather(tx, ty, tz, feat2d, gidx):
    mesh = plsc.VectorSubcoreMesh(core_axis_name="c", subcore_axis_name="s")
    fn = functools.partial(
        pl.kernel,
        mesh=mesh,
        compiler_params=pltpu.CompilerParams(needs_layout_passes=False),
        out_type=[
            jax.ShapeDtypeStruct((BNK,), jnp.float32),
            jax.ShapeDtypeStruct((BNK,), jnp.float32),
            jax.ShapeDtypeStruct((BNK,), jnp.float32),
            jax.ShapeDtypeStruct((2 * BNK, D), jnp.float32),
        ],
        scratch_types=[
            pltpu.VMEM((BN,), jnp.float32),
            pltpu.VMEM((BN,), jnp.float32),
            pltpu.VMEM((BN,), jnp.float32),
            pltpu.VMEM((CH,), jnp.int32),
            pltpu.VMEM((CH,), jnp.int32),
            pltpu.VMEM((CH,), jnp.float32),
            pltpu.VMEM((CH,), jnp.float32),
            pltpu.VMEM((CH,), jnp.float32),
            pltpu.VMEM((CH, D), jnp.float32),
            pltpu.SemaphoreType.DMA,
            pltpu.SemaphoreType.DMA,
            pltpu.SemaphoreType.DMA,
        ],
    )(_sc_gather_body)
    return fn(tx, ty, tz, feat2d, gidx)


def _tc_body(xyz_ref, px_ref, py_ref, pz_ref, w_ref, b_ref, buf_ref, o_ref):
    del buf_ref
    w = w_ref[...]
    wa = w[0:3] - w[6:9]
    wc0 = w[3] + w[6]
    wc1 = w[4] + w[7]
    wc2 = w[5] + w[8]
    w9 = w[9]
    bb = b_ref[...][0]
    wa16 = jnp.concatenate([wa, jnp.zeros((13, D), jnp.float32)], axis=0)

    cen = xyz_ref[...]
    px = px_ref[...]
    py = py_ref[...]
    pz = pz_ref[...]
    dx = px - cen[:, 0:1]
    dy = py - cen[:, 1:2]
    dz = pz - cen[:, 2:3]
    norm = jnp.sqrt(dx * dx + dy * dy + dz * dz)

    dn = (((1,), (0,)), ((), ()))
    cen_a = lax.dot_general(cen, wa16, dn,
                            precision=lax.Precision.HIGHEST)
    r = (cen_a[:, None, :] + bb
         + px[:, :, None] * wc0 + py[:, :, None] * wc1
         + pz[:, :, None] * wc2 + norm[:, :, None] * w9)
    o_ref[...] = jnp.maximum(r, 0.0)


def _tc_assemble(xyz16, px_g, py_g, pz_g, W, b2d, buf):
    grid = (BN // PB,)
    pk_spec = pl.BlockSpec((PB, K), lambda i: (i, 0))
    return pl.pallas_call(
        _tc_body,
        grid=grid,
        in_specs=[
            pl.BlockSpec((PB, 16), lambda i: (i, 0)),
            pk_spec, pk_spec, pk_spec,
            pl.BlockSpec((10, D), lambda i: (0, 0)),
            pl.BlockSpec((1, D), lambda i: (0, 0)),
            pl.BlockSpec(memory_space=pl.ANY),
        ],
        out_specs=pl.BlockSpec((PB, K, D), lambda i: (i, 0, 0)),
        out_shape=jax.ShapeDtypeStruct((BN, K, 2 * D), jnp.float32),
        input_output_aliases={6: 0},
    )(xyz16, px_g, py_g, pz_g, W, b2d, buf)


def kernel(xyz, feat, idx, W, b):
    xyz2 = xyz.reshape(BN, 3)
    xyz16 = jnp.pad(xyz2, ((0, 0), (0, 13)))
    tx = xyz2[:, 0]
    ty = xyz2[:, 1]
    tz = xyz2[:, 2]
    feat2d = feat.reshape(BN, D)
    gidx = (idx + (jnp.arange(B, dtype=idx.dtype) * N)[:, None, None])
    gidx = gidx.reshape(BNK)
    px_g, py_g, pz_g, buf = _sc_gather(tx, ty, tz, feat2d, gidx)
    out = _tc_assemble(xyz16, px_g.reshape(BN, K), py_g.reshape(BN, K),
                       pz_g.reshape(BN, K), W, b.reshape(1, D),
                       buf.reshape(BN, K, 2 * D))
    return out.reshape(B, N, K, 2 * D)


# speedup vs baseline: 1.3185x; 1.3185x over previous
import functools

import jax
import jax.numpy as jnp
from jax import lax
from jax.experimental import pallas as pl
from jax.experimental.pallas import tpu as pltpu
from jax.experimental.pallas import tpu_sc as plsc

B, N, K, D = 4, 4096, 16, 128
BN = B * N
BNK = B * N * K
NW = 32
ROWS_PW = BNK // NW
CH = 512
CH2 = 128
PB = 1024


def _sc_planes_body(tx_hbm, ty_hbm, tz_hbm, gidx_hbm,
                    px_out, py_out, pz_out,
                    txv, tyv, tzv, idxv, pxb, pyb, pzb, sem_i):
    wid = lax.axis_index("s") * 2 + lax.axis_index("c")
    base0 = wid * ROWS_PW
    pltpu.sync_copy(tx_hbm, txv)
    pltpu.sync_copy(ty_hbm, tyv)
    pltpu.sync_copy(tz_hbm, tzv)

    def body(j, carry):
        base = base0 + j * CH
        pltpu.async_copy(gidx_hbm.at[pl.ds(base, CH)], idxv, sem_i).wait()

        def inner(i, c):
            s = pl.ds(i * 16, 16)
            v = idxv[s]
            pxb[s] = plsc.load_gather(txv, [v])
            pyb[s] = plsc.load_gather(tyv, [v])
            pzb[s] = plsc.load_gather(tzv, [v])
            return c

        lax.fori_loop(0, CH // 16, inner, 0)
        pltpu.sync_copy(pxb, px_out.at[pl.ds(base, CH)])
        pltpu.sync_copy(pyb, py_out.at[pl.ds(base, CH)])
        pltpu.sync_copy(pzb, pz_out.at[pl.ds(base, CH)])
        return carry

    lax.fori_loop(0, ROWS_PW // CH, body, 0)


def _sc_planes(tx, ty, tz, gidx):
    mesh = plsc.VectorSubcoreMesh(core_axis_name="c", subcore_axis_name="s")
    fn = functools.partial(
        pl.kernel,
        mesh=mesh,
        compiler_params=pltpu.CompilerParams(needs_layout_passes=False),
        out_type=[
            jax.ShapeDtypeStruct((BNK,), jnp.float32),
            jax.ShapeDtypeStruct((BNK,), jnp.float32),
            jax.ShapeDtypeStruct((BNK,), jnp.float32),
        ],
        scratch_types=[
            pltpu.VMEM((BN,), jnp.float32),
            pltpu.VMEM((BN,), jnp.float32),
            pltpu.VMEM((BN,), jnp.float32),
            pltpu.VMEM((CH,), jnp.int32),
            pltpu.VMEM((CH,), jnp.float32),
            pltpu.VMEM((CH,), jnp.float32),
            pltpu.VMEM((CH,), jnp.float32),
            pltpu.SemaphoreType.DMA,
        ],
    )(_sc_planes_body)
    return fn(tx, ty, tz, gidx)


def _tc_body(xyz_ref, px_ref, py_ref, pz_ref, f_ref, w_ref, b_ref,
             t_ref, g0_ref, nrm_ref):
    w = w_ref[...]
    wa = w[0:3] - w[6:9]
    wc = w[3:6] + w[6:9]
    bb = b_ref[...]
    zpad = jnp.zeros((13, D), jnp.float32)
    wa16 = jnp.concatenate([wa, zpad], axis=0)
    wc16 = jnp.concatenate([wc, zpad], axis=0)

    cen = xyz_ref[...]
    dn = (((1,), (0,)), ((), ()))
    g1 = lax.dot_general(cen, wc16, dn, precision=lax.Precision.HIGHEST)
    g0 = lax.dot_general(cen, wa16, dn, precision=lax.Precision.HIGHEST) + bb
    t_ref[:, 0:D] = g1
    t_ref[:, D:2 * D] = f_ref[...]
    g0_ref[...] = g0

    px = px_ref[...]
    py = py_ref[...]
    pz = pz_ref[...]
    dx = px - cen[:, 0:1]
    dy = py - cen[:, 1:2]
    dz = pz - cen[:, 2:3]
    nrm_ref[...] = jnp.sqrt(dx * dx + dy * dy + dz * dz)


def _tc_tables(xyz16, px_g, py_g, pz_g, feat2d, W, b2d):
    grid = (BN // PB,)
    pk_spec = pl.BlockSpec((PB, K), lambda i: (i, 0))
    return pl.pallas_call(
        _tc_body,
        grid=grid,
        in_specs=[
            pl.BlockSpec((PB, 16), lambda i: (i, 0)),
            pk_spec, pk_spec, pk_spec,
            pl.BlockSpec((PB, D), lambda i: (i, 0)),
            pl.BlockSpec((10, D), lambda i: (0, 0)),
            pl.BlockSpec((1, D), lambda i: (0, 0)),
        ],
        out_specs=[
            pl.BlockSpec((PB, 2 * D), lambda i: (i, 0)),
            pl.BlockSpec((PB, D), lambda i: (i, 0)),
            pk_spec,
        ],
        out_shape=[
            jax.ShapeDtypeStruct((BN, 2 * D), jnp.float32),
            jax.ShapeDtypeStruct((BN, D), jnp.float32),
            jax.ShapeDtypeStruct((BN, K), jnp.float32),
        ],
    )(xyz16, px_g, py_g, pz_g, feat2d, W, b2d)


def _sc_fuse_body(t_hbm, g0_hbm, nrm_hbm, w9_hbm, gidx_hbm, out_hbm,
                  idxv, tbuf, g0b, nrmb, w9v, sem_i, sem_t, sem_o):
    wid = lax.axis_index("s") * 2 + lax.axis_index("c")
    base0 = wid * ROWS_PW
    pltpu.sync_copy(w9_hbm, w9v)
    w9r = [w9v[pl.ds(l * 16, 16)] for l in range(8)]

    def body(j, carry):
        base = base0 + j * CH2
        pltpu.async_copy(gidx_hbm.at[pl.ds(base, CH2)], idxv, sem_i).wait()
        cp_t = pltpu.async_copy(t_hbm.at[idxv], tbuf, sem_t)
        g0_off = pl.multiple_of(base // K, CH2 // K)
        pltpu.sync_copy(g0_hbm.at[pl.ds(g0_off, CH2 // K)], g0b)
        pltpu.sync_copy(nrm_hbm.at[pl.ds(base, CH2)], nrmb)
        cp_t.wait()

        def point(i, c):
            g0r = [g0b[i, pl.ds(l * 16, 16)] for l in range(8)]
            for jj in range(K):
                r = i * K + jj
                nb = plsc.load_gather(nrmb, [jnp.full((16,), r, jnp.int32)])
                for l in range(8):
                    s = pl.ds(l * 16, 16)
                    v = tbuf[r, s] + g0r[l] + nb * w9r[l]
                    tbuf[r, s] = jnp.maximum(v, 0.0)
            return c

        lax.fori_loop(0, CH2 // K, point, 0)
        pltpu.sync_copy(tbuf, out_hbm.at[pl.ds(base, CH2)])
        return carry

    lax.fori_loop(0, ROWS_PW // CH2, body, 0)


def _sc_fuse(t_tab, g0_tab, nrm_flat, w9, gidx):
    mesh = plsc.VectorSubcoreMesh(core_axis_name="c", subcore_axis_name="s")
    fn = functools.partial(
        pl.kernel,
        mesh=mesh,
        compiler_params=pltpu.CompilerParams(needs_layout_passes=False),
        out_type=jax.ShapeDtypeStruct((BNK, 2 * D), jnp.float32),
        scratch_types=[
            pltpu.VMEM((CH2,), jnp.int32),
            pltpu.VMEM((CH2, 2 * D), jnp.float32),
            pltpu.VMEM((CH2 // K, D), jnp.float32),
            pltpu.VMEM((CH2,), jnp.float32),
            pltpu.VMEM((D,), jnp.float32),
            pltpu.SemaphoreType.DMA,
            pltpu.SemaphoreType.DMA,
            pltpu.SemaphoreType.DMA,
        ],
    )(_sc_fuse_body)
    return fn(t_tab, g0_tab, nrm_flat, w9, gidx)


def kernel(xyz, feat, idx, W, b):
    xyz2 = xyz.reshape(BN, 3)
    xyz16 = jnp.pad(xyz2, ((0, 0), (0, 13)))
    tx = xyz2[:, 0]
    ty = xyz2[:, 1]
    tz = xyz2[:, 2]
    feat2d = feat.reshape(BN, D)
    gidx = (idx + (jnp.arange(B, dtype=idx.dtype) * N)[:, None, None])
    gidx = gidx.reshape(BNK)
    px_g, py_g, pz_g = _sc_planes(tx, ty, tz, gidx)
    t_tab, g0_tab, nrm = _tc_tables(xyz16, px_g.reshape(BN, K),
                                    py_g.reshape(BN, K), pz_g.reshape(BN, K),
                                    feat2d, W, b.reshape(1, D))
    out = _sc_fuse(t_tab, g0_tab, nrm.reshape(BNK), W[9], gidx)
    return out.reshape(B, N, K, 2 * D)


# speedup vs baseline: 1.5890x; 1.2052x over previous
import functools

import jax
import jax.numpy as jnp
from jax import lax
from jax.experimental import pallas as pl
from jax.experimental.pallas import tpu as pltpu
from jax.experimental.pallas import tpu_sc as plsc

B, N, K, D = 4, 4096, 16, 128
BN = B * N
BNK = B * N * K
NW = 32
ROWS_PW = BNK // NW
CH2 = 128
NCH = ROWS_PW // CH2
NPAIR = NCH // 2
PB = 1024
SQRT_MAGIC = 0x1FBD1DF5


def _tc_body(xyz_ref, f_ref, w_ref, b_ref, t_ref, g0_ref):
    w = w_ref[...]
    wa = w[0:3] - w[6:9]
    wc = w[3:6] + w[6:9]
    bb = b_ref[...]
    zpad = jnp.zeros((13, D), jnp.float32)
    wa16 = jnp.concatenate([wa, zpad], axis=0)
    wc16 = jnp.concatenate([wc, zpad], axis=0)

    cen = xyz_ref[...]
    dn = (((1,), (0,)), ((), ()))
    g1 = lax.dot_general(cen, wc16, dn, precision=lax.Precision.HIGHEST)
    g0 = lax.dot_general(cen, wa16, dn, precision=lax.Precision.HIGHEST) + bb
    t_ref[:, 0:D] = g1
    t_ref[:, D:2 * D] = f_ref[...]
    g0_ref[...] = g0


def _tc_tables(xyz16, feat2d, W, b2d):
    grid = (BN // PB,)
    return pl.pallas_call(
        _tc_body,
        grid=grid,
        in_specs=[
            pl.BlockSpec((PB, 16), lambda i: (i, 0)),
            pl.BlockSpec((PB, D), lambda i: (i, 0)),
            pl.BlockSpec((10, D), lambda i: (0, 0)),
            pl.BlockSpec((1, D), lambda i: (0, 0)),
        ],
        out_specs=[
            pl.BlockSpec((PB, 2 * D), lambda i: (i, 0)),
            pl.BlockSpec((PB, D), lambda i: (i, 0)),
        ],
        out_shape=[
            jax.ShapeDtypeStruct((BN, 2 * D), jnp.float32),
            jax.ShapeDtypeStruct((BN, D), jnp.float32),
        ],
    )(xyz16, feat2d, W, b2d)


def _sqrt16(x):
    i = plsc.bitcast(x, jnp.int32)
    y = plsc.bitcast((i >> 1) + SQRT_MAGIC, jnp.float32)
    y = 0.5 * (y + x / y)
    y = 0.5 * (y + x / y)
    return y


def _sc_fuse_body(t_hbm, g0_hbm, w9_hbm, tx_hbm, ty_hbm, tz_hbm, gidx_hbm,
                  out_hbm,
                  txv, tyv, tzv, idxa, idxb, tba, tbb, g0b, nrmt, w9v,
                  sem_i, sem_ta, sem_tb, sem_oa, sem_ob):
    wid = lax.axis_index("s") * 2 + lax.axis_index("c")
    base0 = wid * ROWS_PW
    pltpu.sync_copy(tx_hbm, txv)
    pltpu.sync_copy(ty_hbm, tyv)
    pltpu.sync_copy(tz_hbm, tzv)
    pltpu.sync_copy(w9_hbm, w9v)
    w9r = [w9v[pl.ds(l * 16, 16)] for l in range(8)]

    def fuse_chunk(chunk_base, idxv, tbuf):
        g0_off = pl.multiple_of(chunk_base // K, CH2 // K)
        pltpu.sync_copy(g0_hbm.at[pl.ds(g0_off, CH2 // K)], g0b)

        def point(i, c):
            v = idxv[pl.ds(i * K, 16)]
            pxv = plsc.load_gather(txv, [v])
            pyv = plsc.load_gather(tyv, [v])
            pzv = plsc.load_gather(tzv, [v])
            csp = jnp.full((16,), g0_off + i, jnp.int32)
            cxv = plsc.load_gather(txv, [csp])
            cyv = plsc.load_gather(tyv, [csp])
            czv = plsc.load_gather(tzv, [csp])
            dx = pxv - cxv
            dy = pyv - cyv
            dz = pzv - czv
            norm16 = _sqrt16(dx * dx + dy * dy + dz * dz)
            g0r = [g0b[i, pl.ds(l * 16, 16)] for l in range(8)]
            for jj in range(K):
                r = i * K + jj
                nb = lax.gather(
                    norm16, jnp.full((16, 1), jj, jnp.int32),
                    lax.GatherDimensionNumbers(
                        offset_dims=(), collapsed_slice_dims=(0,),
                        start_index_map=(0,)),
                    (1,), mode=lax.GatherScatterMode.PROMISE_IN_BOUNDS)
                for l in range(8):
                    s = pl.ds(l * 16, 16)
                    val = tbuf[r, s] + g0r[l] + nb * w9r[l]
                    tbuf[r, s] = jnp.maximum(val, 0.0)
            return c

        lax.fori_loop(0, CH2 // K, point, 0)

    def pair(jj, carry):
        b0 = base0 + (2 * jj) * CH2
        b1 = b0 + CH2
        pltpu.async_copy(gidx_hbm.at[pl.ds(b0, CH2)], idxa, sem_i).wait()

        @pl.when(jj > 0)
        def _():
            pltpu.make_async_copy(tba, out_hbm.at[pl.ds(b0, CH2)],
                                  sem_oa).wait()

        pltpu.async_copy(t_hbm.at[idxa], tba, sem_ta).wait()
        fuse_chunk(b0, idxa, tba)
        pltpu.async_copy(tba, out_hbm.at[pl.ds(b0, CH2)], sem_oa)

        pltpu.async_copy(gidx_hbm.at[pl.ds(b1, CH2)], idxb, sem_i).wait()

        @pl.when(jj > 0)
        def _():
            pltpu.make_async_copy(tbb, out_hbm.at[pl.ds(b1, CH2)],
                                  sem_ob).wait()

        pltpu.async_copy(t_hbm.at[idxb], tbb, sem_tb).wait()
        fuse_chunk(b1, idxb, tbb)
        pltpu.async_copy(tbb, out_hbm.at[pl.ds(b1, CH2)], sem_ob)
        return carry

    lax.fori_loop(0, NPAIR, pair, 0)
    pltpu.make_async_copy(tba, out_hbm.at[pl.ds(base0, CH2)], sem_oa).wait()
    pltpu.make_async_copy(tbb, out_hbm.at[pl.ds(base0, CH2)], sem_ob).wait()


def _sc_fuse(t_tab, g0_tab, w9, tx, ty, tz, gidx):
    mesh = plsc.VectorSubcoreMesh(core_axis_name="c", subcore_axis_name="s")
    fn = functools.partial(
        pl.kernel,
        mesh=mesh,
        compiler_params=pltpu.CompilerParams(needs_layout_passes=False),
        out_type=jax.ShapeDtypeStruct((BNK, 2 * D), jnp.float32),
        scratch_types=[
            pltpu.VMEM((BN,), jnp.float32),
            pltpu.VMEM((BN,), jnp.float32),
            pltpu.VMEM((BN,), jnp.float32),
            pltpu.VMEM((CH2,), jnp.int32),
            pltpu.VMEM((CH2,), jnp.int32),
            pltpu.VMEM((CH2, 2 * D), jnp.float32),
            pltpu.VMEM((CH2, 2 * D), jnp.float32),
            pltpu.VMEM((CH2 // K, D), jnp.float32),
            pltpu.VMEM((16,), jnp.float32),
            pltpu.VMEM((D,), jnp.float32),
            pltpu.SemaphoreType.DMA,
            pltpu.SemaphoreType.DMA,
            pltpu.SemaphoreType.DMA,
            pltpu.SemaphoreType.DMA,
            pltpu.SemaphoreType.DMA,
        ],
    )(_sc_fuse_body)
    return fn(t_tab, g0_tab, w9, tx, ty, tz, gidx)


def kernel(xyz, feat, idx, W, b):
    xyz2 = xyz.reshape(BN, 3)
    xyz16 = jnp.pad(xyz2, ((0, 0), (0, 13)))
    tx = xyz2[:, 0]
    ty = xyz2[:, 1]
    tz = xyz2[:, 2]
    feat2d = feat.reshape(BN, D)
    gidx = (idx + (jnp.arange(B, dtype=idx.dtype) * N)[:, None, None])
    gidx = gidx.reshape(BNK)
    t_tab, g0_tab = _tc_tables(xyz16, feat2d, W, b.reshape(1, D))
    out = _sc_fuse(t_tab, g0_tab, W[9], tx, ty, tz, gidx)
    return out.reshape(B, N, K, 2 * D)
